# Initial kernel scaffold; baseline (speedup 1.0000x reference)
import jax
import jax.numpy as jnp
from jax.experimental import pallas as pl


def kernel(s, token_to_atom_idx, W, b):
    raise NotImplementedError("write your pallas kernel here")


# speedup vs baseline: 42.6129x; 42.6129x over previous
import jax
import jax.numpy as jnp
from jax.experimental import pallas as pl


def _fused_kernel(s_ref, t_ref, w_ref, b_ref, out_ref):
    h = jax.lax.dot_general(
        s_ref[0], w_ref[...],
        dimension_numbers=(((1,), (1,)), ((), ())),
        preferred_element_type=jnp.float32,
    )
    out_ref[0] = jax.lax.dot_general(
        t_ref[0], h,
        dimension_numbers=(((1,), (0,)), ((), ())),
        preferred_element_type=jnp.float32,
    ) + b_ref[...]


def kernel(s, token_to_atom_idx, W, b):
    *batch, n_res, c_s = s.shape
    n_atom = token_to_atom_idx.shape[-2]
    c_out = W.shape[0]

    s2 = s.reshape(-1, n_res, c_s)
    t2 = token_to_atom_idx.reshape(-1, n_atom, n_res)
    nb = s2.shape[0]
    b2 = b.reshape(1, c_out)

    out = pl.pallas_call(
        _fused_kernel,
        grid=(nb,),
        in_specs=[
            pl.BlockSpec((1, n_res, c_s), lambda i: (i, 0, 0)),
            pl.BlockSpec((1, n_atom, n_res), lambda i: (i, 0, 0)),
            pl.BlockSpec((c_out, c_s), lambda i: (0, 0)),
            pl.BlockSpec((1, c_out), lambda i: (0, 0)),
        ],
        out_specs=pl.BlockSpec((1, n_atom, c_out), lambda i: (i, 0, 0)),
        out_shape=jax.ShapeDtypeStruct((nb, n_atom, c_out), jnp.float32),
    )(s2, t2, W, b2)

    return out.reshape(*batch, n_atom, c_out)
